# baseline (speedup 1.0000x reference)
import functools

import jax
import jax.numpy as jnp
from jax import lax
from jax.experimental import pallas as pl
from jax.experimental.pallas import tpu as pltpu
from jax.experimental.pallas import tpu_sc as plsc

_F8 = jnp.float8_e4m3fn


def _sc_gather(table, idx):
    v, d = table.shape
    b = idx.shape[0]
    info = plsc.get_sparse_core_info()
    nw = info.num_cores * info.num_subcores
    b_per_w = b // nw
    mesh = plsc.VectorSubcoreMesh(core_axis_name="c", subcore_axis_name="s")

    @functools.partial(
        pl.kernel, mesh=mesh,
        out_type=jax.ShapeDtypeStruct((b, d), table.dtype),
        scratch_types=[
            pltpu.VMEM((b_per_w,), jnp.int32),
            pltpu.VMEM((b_per_w, d), table.dtype),
            pltpu.SemaphoreType.DMA,
        ],
    )
    def k(table_hbm, idx_hbm, out_hbm, idx_v, rows_v, sem):
        wid = lax.axis_index("s") * info.num_cores + lax.axis_index("c")
        base = wid * b_per_w
        pltpu.sync_copy(idx_hbm.at[pl.ds(base, b_per_w)], idx_v)
        pltpu.async_copy(table_hbm.at[idx_v], rows_v, sem).wait()
        pltpu.sync_copy(rows_v, out_hbm.at[pl.ds(base, b_per_w)])

    return k(table, idx)


def _mm_body(a_ref, w_ref, o_ref):
    a = a_ref[...].astype(jnp.bfloat16)
    w = w_ref[...].astype(jnp.bfloat16)
    o_ref[...] = jnp.dot(a, w, preferred_element_type=jnp.float32).astype(
        o_ref.dtype)


def _mm(a, w, bm, out_dtype=jnp.bfloat16):
    m, k = a.shape
    n = w.shape[1]
    return pl.pallas_call(
        _mm_body,
        grid=(m // bm,),
        in_specs=[
            pl.BlockSpec((bm, k), lambda i: (i, 0)),
            pl.BlockSpec((k, n), lambda i: (0, 0)),
        ],
        out_specs=pl.BlockSpec((bm, n), lambda i: (i, 0)),
        out_shape=jax.ShapeDtypeStruct((m, n), out_dtype),
        compiler_params=pltpu.CompilerParams(
            dimension_semantics=("arbitrary",)),
    )(a, w)


def _layer1_body(adj_ref, s_ref, b_ref, w2_ref, a8_ref, s2_ref):
    a = adj_ref[...]
    a8_ref[...] = a.astype(_F8)
    acc = jnp.dot(a.astype(jnp.bfloat16), s_ref[...],
                  preferred_element_type=jnp.float32)
    h_blk = jnp.maximum(acc + b_ref[...], 0.0).astype(jnp.bfloat16)
    s2_ref[...] = jnp.dot(h_blk, w2_ref[...],
                          preferred_element_type=jnp.float32).astype(_F8)


def _layer1(adj, s1, b, w2_bf, bm):
    m, k = adj.shape
    n = s1.shape[1]
    return pl.pallas_call(
        _layer1_body,
        grid=(m // bm,),
        in_specs=[
            pl.BlockSpec((bm, k), lambda i: (i, 0)),
            pl.BlockSpec((k, n), lambda i: (0, 0)),
            pl.BlockSpec((1, n), lambda i: (0, 0)),
            pl.BlockSpec((n, n), lambda i: (0, 0)),
        ],
        out_specs=[
            pl.BlockSpec((bm, k), lambda i: (i, 0)),
            pl.BlockSpec((bm, n), lambda i: (i, 0)),
        ],
        out_shape=[
            jax.ShapeDtypeStruct((m, k), _F8),
            jax.ShapeDtypeStruct((m, n), _F8),
        ],
        compiler_params=pltpu.CompilerParams(
            dimension_semantics=("arbitrary",)),
    )(adj, s1, b, w2_bf)


def _layer2_body(adj_ref, s_ref, b_ref, o_ref):
    acc = jnp.dot(adj_ref[...], s_ref[...], preferred_element_type=jnp.float32)
    o_ref[...] = (acc + b_ref[...]).astype(jnp.bfloat16)


def _layer2(adj8, s2, b, bm):
    m, k = adj8.shape
    n = s2.shape[1]
    return pl.pallas_call(
        _layer2_body,
        grid=(m // bm,),
        in_specs=[
            pl.BlockSpec((bm, k), lambda i: (i, 0)),
            pl.BlockSpec((k, n), lambda i: (0, 0)),
            pl.BlockSpec((1, n), lambda i: (0, 0)),
        ],
        out_specs=pl.BlockSpec((bm, n), lambda i: (i, 0)),
        out_shape=jax.ShapeDtypeStruct((m, n), jnp.bfloat16),
        compiler_params=pltpu.CompilerParams(
            dimension_semantics=("arbitrary",)),
    )(adj8, s2, b)


def _head_body(anc_ref, h2_ref, o_ref):
    logits = jax.lax.dot_general(
        anc_ref[...], h2_ref[...],
        (((1,), (1,)), ((), ())), preferred_element_type=jnp.float32)
    o_ref[...] = 0.5 * jnp.tanh(0.5 * logits) + 0.5


def _head(anchors, h2, bm):
    m, n = h2.shape
    nidx = anchors.shape[0]
    return pl.pallas_call(
        _head_body,
        grid=(nidx // bm,),
        in_specs=[
            pl.BlockSpec((bm, n), lambda i: (i, 0)),
            pl.BlockSpec((m, n), lambda i: (0, 0)),
        ],
        out_specs=pl.BlockSpec((bm, m), lambda i: (i, 0)),
        out_shape=jax.ShapeDtypeStruct((nidx, m), jnp.float32),
        compiler_params=pltpu.CompilerParams(
            dimension_semantics=("arbitrary",)),
    )(anchors, h2)


def kernel(x, adj, idx, W1, b1, W2, b2):
    idx32 = idx.astype(jnp.int32)
    b1r = b1.reshape(1, -1)
    b2r = b2.reshape(1, -1)
    w2_bf = W2.astype(jnp.bfloat16)

    support1 = _mm(x, W1, bm=2000)
    adj8, support2 = _layer1(adj, support1, b1r, w2_bf, bm=400)
    h2 = _layer2(adj8, support2, b2r, bm=1000)
    n_nodes, nhid = h2.shape
    h2_i32 = lax.bitcast_convert_type(
        h2.reshape(n_nodes, nhid // 2, 2), jnp.int32)
    anc_i32 = _sc_gather(h2_i32, idx32)
    anchors = lax.bitcast_convert_type(
        anc_i32, jnp.bfloat16).reshape(idx32.shape[0], nhid)
    return _head(anchors, h2, bm=256)


# speedup vs baseline: 1.6509x; 1.6509x over previous
import functools

import jax
import jax.numpy as jnp
from jax.experimental import pallas as pl
from jax.experimental.pallas import tpu as pltpu

_F8 = jnp.float8_e4m3fn


def _mm_body(a_ref, w_ref, o_ref):
    a = a_ref[...].astype(jnp.bfloat16)
    w = w_ref[...].astype(jnp.bfloat16)
    o_ref[...] = jnp.dot(a, w, preferred_element_type=jnp.float32).astype(
        o_ref.dtype)


def _mm(a, w, bm, out_dtype=jnp.bfloat16):
    m, k = a.shape
    n = w.shape[1]
    return pl.pallas_call(
        _mm_body,
        grid=(m // bm,),
        in_specs=[
            pl.BlockSpec((bm, k), lambda i: (i, 0)),
            pl.BlockSpec((k, n), lambda i: (0, 0)),
        ],
        out_specs=pl.BlockSpec((bm, n), lambda i: (i, 0)),
        out_shape=jax.ShapeDtypeStruct((m, n), out_dtype),
        compiler_params=pltpu.CompilerParams(
            dimension_semantics=("arbitrary",)),
    )(a, w)


def _layer1_body(adj_ref, s_ref, b_ref, w2_ref, a8_ref, s2_ref):
    a = adj_ref[...]
    a8_ref[...] = a.astype(_F8)
    acc = jnp.dot(a.astype(jnp.bfloat16), s_ref[...],
                  preferred_element_type=jnp.float32)
    h_blk = jnp.maximum(acc + b_ref[...], 0.0).astype(jnp.bfloat16)
    s2_ref[...] = jnp.dot(h_blk, w2_ref[...],
                          preferred_element_type=jnp.float32).astype(_F8)


def _layer1(adj, s1, b, w2_bf, bm):
    m, k = adj.shape
    n = s1.shape[1]
    return pl.pallas_call(
        _layer1_body,
        grid=(m // bm,),
        in_specs=[
            pl.BlockSpec((bm, k), lambda i: (i, 0)),
            pl.BlockSpec((k, n), lambda i: (0, 0)),
            pl.BlockSpec((1, n), lambda i: (0, 0)),
            pl.BlockSpec((n, n), lambda i: (0, 0)),
        ],
        out_specs=[
            pl.BlockSpec((bm, k), lambda i: (i, 0)),
            pl.BlockSpec((bm, n), lambda i: (i, 0)),
        ],
        out_shape=[
            jax.ShapeDtypeStruct((m, k), _F8),
            jax.ShapeDtypeStruct((m, n), _F8),
        ],
        compiler_params=pltpu.CompilerParams(
            dimension_semantics=("arbitrary",)),
    )(adj, s1, b, w2_bf)


def _layer2_body(adj_ref, s_ref, b_ref, o_ref):
    acc = jnp.dot(adj_ref[...], s_ref[...], preferred_element_type=jnp.float32)
    o_ref[...] = (acc + b_ref[...]).astype(jnp.bfloat16)


def _layer2(adj8, s2, b, bm):
    m, k = adj8.shape
    n = s2.shape[1]
    return pl.pallas_call(
        _layer2_body,
        grid=(m // bm,),
        in_specs=[
            pl.BlockSpec((bm, k), lambda i: (i, 0)),
            pl.BlockSpec((k, n), lambda i: (0, 0)),
            pl.BlockSpec((1, n), lambda i: (0, 0)),
        ],
        out_specs=pl.BlockSpec((bm, n), lambda i: (i, 0)),
        out_shape=jax.ShapeDtypeStruct((m, n), jnp.bfloat16),
        compiler_params=pltpu.CompilerParams(
            dimension_semantics=("arbitrary",)),
    )(adj8, s2, b)


def _head_body(idx_ref, h2_ref, o_ref, *, bk):
    m = h2_ref.shape[0]
    bm = idx_ref.shape[0]
    nk = m // bk
    acc = jnp.zeros((bm, h2_ref.shape[1]), jnp.float32)
    for kk in range(nk):
        cols = kk * bk + jax.lax.broadcasted_iota(jnp.int32, (bm, bk), 1)
        e = (idx_ref[...] == cols).astype(jnp.bfloat16)
        acc = acc + jnp.dot(e, h2_ref[kk * bk:(kk + 1) * bk, :],
                            preferred_element_type=jnp.float32)
    anchors = acc.astype(jnp.bfloat16)
    logits = jax.lax.dot_general(
        anchors, h2_ref[...],
        (((1,), (1,)), ((), ())), preferred_element_type=jnp.float32)
    o_ref[...] = 0.5 * jnp.tanh(0.5 * logits) + 0.5


def _head(h2, idx2d, bm, bk):
    m, n = h2.shape
    nidx = idx2d.shape[0]
    return pl.pallas_call(
        functools.partial(_head_body, bk=bk),
        grid=(nidx // bm,),
        in_specs=[
            pl.BlockSpec((bm, 1), lambda i: (i, 0)),
            pl.BlockSpec((m, n), lambda i: (0, 0)),
        ],
        out_specs=pl.BlockSpec((bm, m), lambda i: (i, 0)),
        out_shape=jax.ShapeDtypeStruct((nidx, m), jnp.float32),
        compiler_params=pltpu.CompilerParams(
            dimension_semantics=("arbitrary",)),
    )(idx2d, h2)


def kernel(x, adj, idx, W1, b1, W2, b2):
    idx32 = idx.astype(jnp.int32)
    b1r = b1.reshape(1, -1)
    b2r = b2.reshape(1, -1)
    w2_bf = W2.astype(jnp.bfloat16)

    support1 = _mm(x, W1, bm=2000)
    adj8, support2 = _layer1(adj, support1, b1r, w2_bf, bm=400)
    h2 = _layer2(adj8, support2, b2r, bm=1000)
    return _head(h2, idx32.reshape(-1, 1), bm=128, bk=2000)


# speedup vs baseline: 1.7017x; 1.0308x over previous
import functools

import jax
import jax.numpy as jnp
from jax.experimental import pallas as pl
from jax.experimental.pallas import tpu as pltpu

_F8 = jnp.float8_e4m3fn


def _mm_body(a_ref, w_ref, o_ref):
    a = a_ref[...].astype(jnp.bfloat16)
    w = w_ref[...].astype(jnp.bfloat16)
    o_ref[...] = jnp.dot(a, w, preferred_element_type=jnp.float32).astype(
        o_ref.dtype)


def _mm(a, w, bm, out_dtype=jnp.bfloat16):
    m, k = a.shape
    n = w.shape[1]
    return pl.pallas_call(
        _mm_body,
        grid=(m // bm,),
        in_specs=[
            pl.BlockSpec((bm, k), lambda i: (i, 0)),
            pl.BlockSpec((k, n), lambda i: (0, 0)),
        ],
        out_specs=pl.BlockSpec((bm, n), lambda i: (i, 0)),
        out_shape=jax.ShapeDtypeStruct((m, n), out_dtype),
        compiler_params=pltpu.CompilerParams(
            dimension_semantics=("arbitrary",)),
    )(a, w)


def _layer1_body(adj_ref, s_ref, b_ref, w2_ref, a8_ref, s2_ref):
    a = adj_ref[...]
    a8_ref[...] = a.astype(_F8)
    acc = jnp.dot(a.astype(jnp.bfloat16), s_ref[...],
                  preferred_element_type=jnp.float32)
    h_blk = jnp.maximum(acc + b_ref[...], 0.0).astype(jnp.bfloat16)
    s2_ref[...] = jnp.dot(h_blk, w2_ref[...],
                          preferred_element_type=jnp.float32).astype(_F8)


def _layer1(adj, s1, b, w2_bf, bm):
    m, k = adj.shape
    n = s1.shape[1]
    return pl.pallas_call(
        _layer1_body,
        grid=(m // bm,),
        in_specs=[
            pl.BlockSpec((bm, k), lambda i: (i, 0)),
            pl.BlockSpec((k, n), lambda i: (0, 0)),
            pl.BlockSpec((1, n), lambda i: (0, 0)),
            pl.BlockSpec((n, n), lambda i: (0, 0)),
        ],
        out_specs=[
            pl.BlockSpec((bm, k), lambda i: (i, 0)),
            pl.BlockSpec((bm, n), lambda i: (i, 0)),
        ],
        out_shape=[
            jax.ShapeDtypeStruct((m, k), _F8),
            jax.ShapeDtypeStruct((m, n), _F8),
        ],
        compiler_params=pltpu.CompilerParams(
            dimension_semantics=("arbitrary",)),
    )(adj, s1, b, w2_bf)


def _layer2_body(adj_ref, s_ref, b_ref, o_ref):
    acc = jnp.dot(adj_ref[...], s_ref[...], preferred_element_type=jnp.float32)
    o_ref[...] = (acc + b_ref[...]).astype(jnp.bfloat16)


def _layer2(adj8, s2, b, bm):
    m, k = adj8.shape
    n = s2.shape[1]
    return pl.pallas_call(
        _layer2_body,
        grid=(m // bm,),
        in_specs=[
            pl.BlockSpec((bm, k), lambda i: (i, 0)),
            pl.BlockSpec((k, n), lambda i: (0, 0)),
            pl.BlockSpec((1, n), lambda i: (0, 0)),
        ],
        out_specs=pl.BlockSpec((bm, n), lambda i: (i, 0)),
        out_shape=jax.ShapeDtypeStruct((m, n), jnp.bfloat16),
        compiler_params=pltpu.CompilerParams(
            dimension_semantics=("arbitrary",)),
    )(adj8, s2, b)


def _head_body(idx_ref, h2_ref, o_ref, *, bk):
    m = h2_ref.shape[0]
    bm = idx_ref.shape[0]
    nk = m // bk
    acc = jnp.zeros((bm, h2_ref.shape[1]), jnp.float32)
    for kk in range(nk):
        cols = kk * bk + jax.lax.broadcasted_iota(jnp.int32, (bm, bk), 1)
        e = (idx_ref[...] == cols).astype(jnp.bfloat16)
        acc = acc + jnp.dot(e, h2_ref[kk * bk:(kk + 1) * bk, :],
                            preferred_element_type=jnp.float32)
    anchors = acc.astype(jnp.bfloat16)
    logits = jax.lax.dot_general(
        anchors, h2_ref[...],
        (((1,), (1,)), ((), ())), preferred_element_type=jnp.float32)
    o_ref[...] = 0.5 * jnp.tanh(0.5 * logits) + 0.5


def _head(h2, idx2d, bm, bk):
    m, n = h2.shape
    nidx = idx2d.shape[0]
    return pl.pallas_call(
        functools.partial(_head_body, bk=bk),
        grid=(nidx // bm,),
        in_specs=[
            pl.BlockSpec((bm, 1), lambda i: (i, 0)),
            pl.BlockSpec((m, n), lambda i: (0, 0)),
        ],
        out_specs=pl.BlockSpec((bm, m), lambda i: (i, 0)),
        out_shape=jax.ShapeDtypeStruct((nidx, m), jnp.float32),
        compiler_params=pltpu.CompilerParams(
            dimension_semantics=("arbitrary",)),
    )(idx2d, h2)


def kernel(x, adj, idx, W1, b1, W2, b2):
    idx32 = idx.astype(jnp.int32)
    b1r = b1.reshape(1, -1)
    b2r = b2.reshape(1, -1)
    w2_bf = W2.astype(jnp.bfloat16)

    support1 = _mm(x, W1, bm=2000)
    adj8, support2 = _layer1(adj, support1, b1r, w2_bf, bm=400)
    h2 = _layer2(adj8, support2, b2r, bm=1000)
    return _head(h2, idx32.reshape(-1, 1), bm=256, bk=10000)


# speedup vs baseline: 1.7167x; 1.0089x over previous
import functools

import jax
import jax.numpy as jnp
from jax.experimental import pallas as pl
from jax.experimental.pallas import tpu as pltpu

_F8 = jnp.float8_e4m3fn


def _mm_body(a_ref, w_ref, o_ref):
    a = a_ref[...].astype(jnp.bfloat16)
    w = w_ref[...].astype(jnp.bfloat16)
    o_ref[...] = jnp.dot(a, w, preferred_element_type=jnp.float32).astype(
        o_ref.dtype)


def _mm(a, w, bm, out_dtype=jnp.bfloat16):
    m, k = a.shape
    n = w.shape[1]
    return pl.pallas_call(
        _mm_body,
        grid=(m // bm,),
        in_specs=[
            pl.BlockSpec((bm, k), lambda i: (i, 0)),
            pl.BlockSpec((k, n), lambda i: (0, 0)),
        ],
        out_specs=pl.BlockSpec((bm, n), lambda i: (i, 0)),
        out_shape=jax.ShapeDtypeStruct((m, n), out_dtype),
        compiler_params=pltpu.CompilerParams(
            dimension_semantics=("arbitrary",)),
    )(a, w)


def _layer1_body(adj_ref, s_ref, b_ref, w2_ref, a8_ref, s2_ref):
    a = adj_ref[...]
    a8_ref[...] = a.astype(_F8)
    acc = jnp.dot(a.astype(jnp.bfloat16), s_ref[...],
                  preferred_element_type=jnp.float32)
    h_blk = jnp.maximum(acc + b_ref[...], 0.0).astype(jnp.bfloat16)
    s2_ref[...] = jnp.dot(h_blk, w2_ref[...],
                          preferred_element_type=jnp.float32).astype(_F8)


def _layer1(adj, s1, b, w2_bf, bm):
    m, k = adj.shape
    n = s1.shape[1]
    return pl.pallas_call(
        _layer1_body,
        grid=(m // bm,),
        in_specs=[
            pl.BlockSpec((bm, k), lambda i: (i, 0)),
            pl.BlockSpec((k, n), lambda i: (0, 0)),
            pl.BlockSpec((1, n), lambda i: (0, 0)),
            pl.BlockSpec((n, n), lambda i: (0, 0)),
        ],
        out_specs=[
            pl.BlockSpec((bm, k), lambda i: (i, 0)),
            pl.BlockSpec((bm, n), lambda i: (i, 0)),
        ],
        out_shape=[
            jax.ShapeDtypeStruct((m, k), _F8),
            jax.ShapeDtypeStruct((m, n), _F8),
        ],
        compiler_params=pltpu.CompilerParams(
            dimension_semantics=("arbitrary",)),
    )(adj, s1, b, w2_bf)


def _layer2_body(adj_ref, s_ref, b_ref, o_ref):
    acc = jnp.dot(adj_ref[...], s_ref[...], preferred_element_type=jnp.float32)
    o_ref[...] = (acc + b_ref[...]).astype(jnp.bfloat16)


def _layer2(adj8, s2, b, bm):
    m, k = adj8.shape
    n = s2.shape[1]
    return pl.pallas_call(
        _layer2_body,
        grid=(m // bm,),
        in_specs=[
            pl.BlockSpec((bm, k), lambda i: (i, 0)),
            pl.BlockSpec((k, n), lambda i: (0, 0)),
            pl.BlockSpec((1, n), lambda i: (0, 0)),
        ],
        out_specs=pl.BlockSpec((bm, n), lambda i: (i, 0)),
        out_shape=jax.ShapeDtypeStruct((m, n), jnp.bfloat16),
        compiler_params=pltpu.CompilerParams(
            dimension_semantics=("arbitrary",)),
    )(adj8, s2, b)


def _head_body(idx_ref, h2_ref, o_ref, *, bk):
    m = h2_ref.shape[0]
    bm = idx_ref.shape[0]
    nk = m // bk
    acc = jnp.zeros((bm, h2_ref.shape[1]), jnp.float32)
    for kk in range(nk):
        cols = kk * bk + jax.lax.broadcasted_iota(jnp.int32, (bm, bk), 1)
        e = (idx_ref[...] == cols).astype(jnp.bfloat16)
        acc = acc + jnp.dot(e, h2_ref[kk * bk:(kk + 1) * bk, :],
                            preferred_element_type=jnp.float32)
    anchors = acc.astype(jnp.bfloat16)
    logits = jax.lax.dot_general(
        anchors, h2_ref[...],
        (((1,), (1,)), ((), ())), preferred_element_type=jnp.float32)
    o_ref[...] = 0.5 * jnp.tanh(0.5 * logits) + 0.5


def _head(h2, idx2d, bm, bk):
    m, n = h2.shape
    nidx = idx2d.shape[0]
    return pl.pallas_call(
        functools.partial(_head_body, bk=bk),
        grid=(nidx // bm,),
        in_specs=[
            pl.BlockSpec((bm, 1), lambda i: (i, 0)),
            pl.BlockSpec((m, n), lambda i: (0, 0)),
        ],
        out_specs=pl.BlockSpec((bm, m), lambda i: (i, 0)),
        out_shape=jax.ShapeDtypeStruct((nidx, m), jnp.float32),
        compiler_params=pltpu.CompilerParams(
            dimension_semantics=("arbitrary",)),
    )(idx2d, h2)


def kernel(x, adj, idx, W1, b1, W2, b2):
    idx32 = idx.astype(jnp.int32)
    b1r = b1.reshape(1, -1)
    b2r = b2.reshape(1, -1)
    w2_bf = W2.astype(jnp.bfloat16)

    support1 = _mm(x, W1, bm=2000)
    adj8, support2 = _layer1(adj, support1, b1r, w2_bf, bm=400)
    h2 = _layer2(adj8, support2, b2r, bm=1000)
    return _head(h2, idx32.reshape(-1, 1), bm=256, bk=2000)
